# baseline (speedup 1.0000x reference)
import functools

import jax
import jax.numpy as jnp
from jax import lax
from jax.experimental import pallas as pl
from jax.experimental.pallas import tpu as pltpu
from jax.experimental.pallas import tpu_sc as plsc

B = 4096
F = 26
V = 100000
D = 16
DD = 13

NC = 2
NS = 16
NW = NC * NS
ROWS = B // NW
NIDX = ROWS * F
FP = 32
G = 128


def _sc_body(o2_hbm, lin_hbm, idxo_hbm, idxl_hbm, s_hbm, v2_hbm,
             idxo_v, idxl_v, rows_v, lin_v, s_v, v2_v, sem_g, sem_l):
    wid = lax.axis_index("s") * NC + lax.axis_index("c")
    base = wid * ROWS

    pltpu.sync_copy(idxo_hbm.at[wid], idxo_v)
    pltpu.sync_copy(idxl_hbm.at[wid], idxl_v)

    o2_copies = []
    for k in range(F):
        o2_copies.append(pltpu.async_copy(
            o2_hbm.at[idxo_v.at[k]], rows_v.at[pl.ds(k * G, G)], sem_g))
    lin_copies = []
    for k in range(FP):
        lin_copies.append(pltpu.async_copy(
            lin_hbm.at[idxl_v.at[k]], lin_v.at[pl.ds(k * G, G)], sem_l))
    for c in o2_copies:
        c.wait()
    for c in lin_copies:
        c.wait()

    lane = lax.iota(jnp.int32, 16)
    mask2 = jnp.where(lane < (F - 16), 1.0, 0.0).astype(jnp.float32)

    def row_body(r, carry):
        s = jnp.zeros((16,), jnp.float32)
        qv = jnp.zeros((16,), jnp.float32)
        for j in range(F):
            e = rows_v[r * F + j]
            s = s + e
            qv = qv + e * e
        l1 = lin_v[pl.ds(r * FP, 16)]
        l2 = lin_v[pl.ds(r * FP + 16, 16)]
        v2_v[r] = 0.5 * (s * s - qv) + l1 + l2 * mask2
        s_v[r] = s
        return carry

    lax.fori_loop(0, ROWS, row_body, 0)

    pltpu.sync_copy(s_v, s_hbm.at[pl.ds(base, ROWS)])
    pltpu.sync_copy(v2_v, v2_hbm.at[pl.ds(base, ROWS)])


_sc_gather = pl.kernel(
    _sc_body,
    out_type=(
        jax.ShapeDtypeStruct((B, D), jnp.float32),
        jax.ShapeDtypeStruct((B, D), jnp.float32),
    ),
    mesh=plsc.VectorSubcoreMesh(
        core_axis_name="c", subcore_axis_name="s", num_cores=NC,
        num_subcores=NS),
    scratch_types=[
        pltpu.VMEM((F, G), jnp.int32),
        pltpu.VMEM((FP, G), jnp.int32),
        pltpu.VMEM((NIDX, D), jnp.float32),
        pltpu.VMEM((ROWS * FP,), jnp.float32),
        pltpu.VMEM((ROWS, D), jnp.float32),
        pltpu.VMEM((ROWS, D), jnp.float32),
        pltpu.SemaphoreType.DMA,
        pltpu.SemaphoreType.DMA,
    ],
    compiler_params=pltpu.CompilerParams(use_tc_tiling_on_sc=False),
)


def _tc_body(xd_ref, s_ref, v2_ref, w2_ref, b2_ref, dw_ref, db_ref, o_ref):
    xd = xd_ref[...]
    W = w2_ref[...]
    bb = b2_ref[...]
    s_de = jnp.dot(xd, W, preferred_element_type=jnp.float32)
    s_de = s_de + jnp.sum(bb, axis=0, keepdims=True)
    sw2 = jnp.sum(W * W, axis=1, keepdims=True)
    swb = jnp.sum(W * bb, axis=1, keepdims=True)
    sb2 = jnp.sum(bb * bb, keepdims=True)
    q_de = (jnp.dot(xd * xd, sw2, preferred_element_type=jnp.float32)
            + 2.0 * jnp.dot(xd, swb, preferred_element_type=jnp.float32)
            + sb2)
    lin_de = jnp.dot(xd, dw_ref[...], preferred_element_type=jnp.float32)
    s_sp = s_ref[...]
    o_ref[...] = (jnp.sum(v2_ref[...], axis=1, keepdims=True)
                  + jnp.sum(s_sp * s_de, axis=1, keepdims=True)
                  + 0.5 * (jnp.sum(s_de * s_de, axis=1, keepdims=True) - q_de)
                  + lin_de + db_ref[...])


_tc_combine = pl.pallas_call(
    _tc_body,
    out_shape=jax.ShapeDtypeStruct((B, 1), jnp.float32),
)


def kernel(x_sparse, x_dense, lin_tables, o2_tables, dense_W, dense_b,
           o2d_W, o2d_b):
    o2_flat = o2_tables.reshape(F * V, D)
    lin_flat = lin_tables.reshape(F * V)
    flat_idx = x_sparse + (jnp.arange(F, dtype=jnp.int32) * V)[None, :]
    idxo = flat_idx.reshape(NW, F, G)
    idxl = jnp.pad(flat_idx, ((0, 0), (0, FP - F))).reshape(NW, FP, G)
    s_all, v2_all = _sc_gather(o2_flat, lin_flat, idxo, idxl)
    return _tc_combine(x_dense, s_all, v2_all, o2d_W, o2d_b, dense_W,
                       dense_b.reshape(1, 1))


# speedup vs baseline: 1.1106x; 1.1106x over previous
import jax
import jax.numpy as jnp
from jax import lax
from jax.experimental import pallas as pl
from jax.experimental.pallas import tpu as pltpu
from jax.experimental.pallas import tpu_sc as plsc

B = 4096
F = 26
V = 100000
D = 16
DD = 13

NC = 2
NS = 16
NW = NC * NS
ROWS = B // NW


def _sc_body(o2_hbm, lin_hbm, idx_hbm, s_hbm, v2_hbm,
             idx_v, rows_v, lin_v, s_v, v2_v, sem_g, sem_l):
    wid = lax.axis_index("s") * NC + lax.axis_index("c")
    base = wid * ROWS

    pltpu.sync_copy(idx_hbm.at[wid], idx_v)

    o2_copies = []
    lin_copies = []
    for f in range(F):
        o2_copies.append(pltpu.async_copy(
            o2_hbm.at[f].at[idx_v.at[f]],
            rows_v.at[pl.ds(f * ROWS, ROWS)], sem_g))
        lin_copies.append(pltpu.async_copy(
            lin_hbm.at[f].at[idx_v.at[f]],
            lin_v.at[pl.ds(f * ROWS, ROWS)], sem_l))
    for c in o2_copies:
        c.wait()
    for c in lin_copies:
        c.wait()

    def row_body(r, carry):
        s = jnp.zeros((16,), jnp.float32)
        qv = jnp.zeros((16,), jnp.float32)
        for f in range(F):
            e = rows_v[f * ROWS + r]
            s = s + e
            qv = qv + e * e
        v2_v[pl.ds(r * D, 16)] = 0.5 * (s * s - qv)
        s_v[r] = s
        return carry

    lax.fori_loop(0, ROWS, row_body, 0)

    lanes = lax.iota(jnp.int32, 16)

    def grp_body(g, carry):
        lacc = jnp.zeros((16,), jnp.float32)
        for f in range(F):
            lacc = lacc + lin_v[pl.ds(f * ROWS + g * 16, 16)]
        plsc.addupdate_scatter(v2_v, [lanes * D + (g * 16 * D)], lacc)
        return carry

    lax.fori_loop(0, ROWS // 16, grp_body, 0)

    pltpu.sync_copy(s_v, s_hbm.at[pl.ds(base, ROWS)])
    pltpu.sync_copy(v2_v, v2_hbm.at[pl.ds(base * D, ROWS * D)])


_sc_gather = pl.kernel(
    _sc_body,
    out_type=(
        jax.ShapeDtypeStruct((B, D), jnp.float32),
        jax.ShapeDtypeStruct((B * D,), jnp.float32),
    ),
    mesh=plsc.VectorSubcoreMesh(
        core_axis_name="c", subcore_axis_name="s", num_cores=NC,
        num_subcores=NS),
    scratch_types=[
        pltpu.VMEM((F, ROWS), jnp.int32),
        pltpu.VMEM((F * ROWS, D), jnp.float32),
        pltpu.VMEM((F * ROWS,), jnp.float32),
        pltpu.VMEM((ROWS, D), jnp.float32),
        pltpu.VMEM((ROWS * D,), jnp.float32),
        pltpu.SemaphoreType.DMA,
        pltpu.SemaphoreType.DMA,
    ],
    compiler_params=pltpu.CompilerParams(use_tc_tiling_on_sc=False,
                                         needs_layout_passes=False),
)


def _tc_body(xd_ref, s_ref, v2_ref, w2_ref, b2_ref, dw_ref, db_ref, o_ref):
    xd = xd_ref[...]
    W = w2_ref[...]
    bb = b2_ref[...]
    s_de = jnp.dot(xd, W, preferred_element_type=jnp.float32)
    s_de = s_de + jnp.sum(bb, axis=0, keepdims=True)
    sw2 = jnp.sum(W * W, axis=1, keepdims=True)
    swb = jnp.sum(W * bb, axis=1, keepdims=True)
    sb2 = jnp.sum(bb * bb, keepdims=True)
    q_de = (jnp.dot(xd * xd, sw2, preferred_element_type=jnp.float32)
            + 2.0 * jnp.dot(xd, swb, preferred_element_type=jnp.float32)
            + sb2)
    lin_de = jnp.dot(xd, dw_ref[...], preferred_element_type=jnp.float32)
    s_sp = s_ref[...]
    o_ref[...] = (jnp.sum(v2_ref[...], axis=1, keepdims=True)
                  + jnp.sum(s_sp * s_de, axis=1, keepdims=True)
                  + 0.5 * (jnp.sum(s_de * s_de, axis=1, keepdims=True) - q_de)
                  + lin_de + db_ref[...])


_tc_combine = pl.pallas_call(
    _tc_body,
    out_shape=jax.ShapeDtypeStruct((B, 1), jnp.float32),
)


def kernel(x_sparse, x_dense, lin_tables, o2_tables, dense_W, dense_b,
           o2d_W, o2d_b):
    lin2 = lin_tables.reshape(F, V)
    idx = x_sparse.T.reshape(F, NW, ROWS).transpose(1, 0, 2)
    s_all, v2_flat = _sc_gather(o2_tables, lin2, idx)
    v2_all = v2_flat.reshape(B, D)
    return _tc_combine(x_dense, s_all, v2_all, o2d_W, o2d_b, dense_W,
                       dense_b.reshape(1, 1))
